# baseline (speedup 1.0000x reference)
import functools

import jax
import jax.numpy as jnp
from jax import lax
from jax.experimental import pallas as pl
from jax.experimental.pallas import tpu as pltpu
from jax.experimental.pallas import tpu_sc as plsc

B = 16384
D = 128
L = 16
NC = 2
NS = 16
NW = NC * NS
BPW = B // NW
C = 256
NCHUNK = BPW // C


def _sig(x):
    return 1.0 / (1.0 + jnp.exp(-x))


_mesh = plsc.VectorSubcoreMesh(core_axis_name="c", subcore_axis_name="s")


@functools.partial(
    pl.kernel,
    mesh=_mesh,
    out_type=jax.ShapeDtypeStruct((B,), jnp.float32),
    compiler_params=pltpu.CompilerParams(needs_layout_passes=False),
    scratch_types=[
        pltpu.VMEM((BPW,), jnp.int32),
        pltpu.VMEM((BPW,), jnp.int32),
        pltpu.VMEM((C, D), jnp.float32),
        pltpu.VMEM((C, D), jnp.float32),
        pltpu.VMEM((C, D), jnp.float32),
        pltpu.VMEM((BPW,), jnp.float32),
        pltpu.VMEM((BPW,), jnp.float32),
        pltpu.SemaphoreType.DMA,
        pltpu.SemaphoreType.DMA,
        pltpu.SemaphoreType.DMA,
        pltpu.SemaphoreType.DMA,
    ],
)
def _if_sc(sid_hbm, qid_hbm, q_hbm, se_hbm, df_hbm, disc_hbm, out_hbm,
           sid_v, qid_v, se_v, df_v, q_v, disc_v, out_v,
           sem_se, sem_df, sem_q, sem_disc):
    wid = lax.axis_index("s") * NC + lax.axis_index("c")
    base = wid * BPW

    pltpu.sync_copy(sid_hbm.at[pl.ds(base, BPW)], sid_v)
    pltpu.sync_copy(qid_hbm.at[pl.ds(base, BPW)], qid_v)
    cp_disc = pltpu.async_copy(disc_hbm.at[qid_v], disc_v, sem_disc)

    lanes = lax.iota(jnp.int32, L)

    for c in range(NCHUNK):
        cp_se = pltpu.async_copy(se_hbm.at[sid_v.at[pl.ds(c * C, C)]], se_v, sem_se)
        cp_df = pltpu.async_copy(df_hbm.at[qid_v.at[pl.ds(c * C, C)]], df_v, sem_df)
        cp_q = pltpu.async_copy(q_hbm.at[pl.ds(base + c * C, C)], q_v, sem_q)
        cp_se.wait()
        cp_df.wait()
        cp_q.wait()
        if c == 0:
            cp_disc.wait()

        def grp_body(g, _, c=c):
            gtot = jnp.zeros((L,), jnp.float32)
            for r in range(L):
                row = g * L + r
                acc = jnp.zeros((L,), jnp.float32)
                for j in range(D // L):
                    sl = pl.ds(j * L, L)
                    se = se_v[row, sl]
                    df = df_v[row, sl]
                    q = q_v[row, sl]
                    acc = acc + (_sig(se) - _sig(df)) * q
                gtot = jnp.where(lanes == r, jnp.sum(acc), gtot)
            sl = pl.ds(c * C + g * L, L)
            dsc = disc_v[sl]
            out_v[sl] = _sig(_sig(dsc) * gtot)
            return _

        lax.fori_loop(0, C // L, grp_body, 0)

    pltpu.sync_copy(out_v, out_hbm.at[pl.ds(base, BPW)])


def kernel(student_id, question, q_matrix_line, student_emb, difficulty, discrimination):
    sid = student_id.astype(jnp.int32)
    qid = question.astype(jnp.int32)
    return _if_sc(sid, qid, q_matrix_line, student_emb, difficulty,
                  discrimination.reshape(-1))


# speedup vs baseline: 2.3067x; 2.3067x over previous
import functools

import jax
import jax.numpy as jnp
from jax import lax
from jax.experimental import pallas as pl
from jax.experimental.pallas import tpu as pltpu
from jax.experimental.pallas import tpu_sc as plsc

B = 16384
D = 128
NC = 2
NS = 16
NW = NC * NS
BPW = B // NW
C = 256
NCHUNK = BPW // C

_mesh = plsc.VectorSubcoreMesh(core_axis_name="c", subcore_axis_name="s")


@functools.partial(
    pl.kernel,
    mesh=_mesh,
    out_type=(
        jax.ShapeDtypeStruct((B, D), jnp.float32),
        jax.ShapeDtypeStruct((B, D), jnp.float32),
        jax.ShapeDtypeStruct((B,), jnp.float32),
    ),
    compiler_params=pltpu.CompilerParams(needs_layout_passes=False),
    scratch_types=[
        pltpu.VMEM((BPW,), jnp.int32),
        pltpu.VMEM((BPW,), jnp.int32),
        pltpu.VMEM((C, D), jnp.float32),
        pltpu.VMEM((C, D), jnp.float32),
        pltpu.VMEM((BPW,), jnp.float32),
        pltpu.SemaphoreType.DMA,
        pltpu.SemaphoreType.DMA,
        pltpu.SemaphoreType.DMA,
    ],
)
def _gather_sc(sid_hbm, qid_hbm, se_hbm, df_hbm, disc_hbm,
               seg_hbm, dfg_hbm, discg_hbm,
               sid_v, qid_v, se_v, df_v, disc_v,
               sem_se, sem_df, sem_disc):
    wid = lax.axis_index("s") * NC + lax.axis_index("c")
    base = wid * BPW

    pltpu.sync_copy(sid_hbm.at[pl.ds(base, BPW)], sid_v)
    pltpu.sync_copy(qid_hbm.at[pl.ds(base, BPW)], qid_v)
    cp_disc = pltpu.async_copy(disc_hbm.at[qid_v], disc_v, sem_disc)

    for c in range(NCHUNK):
        cp_se = pltpu.async_copy(se_hbm.at[sid_v.at[pl.ds(c * C, C)]], se_v, sem_se)
        cp_df = pltpu.async_copy(df_hbm.at[qid_v.at[pl.ds(c * C, C)]], df_v, sem_df)
        cp_se.wait()
        pltpu.sync_copy(se_v, seg_hbm.at[pl.ds(base + c * C, C)])
        cp_df.wait()
        pltpu.sync_copy(df_v, dfg_hbm.at[pl.ds(base + c * C, C)])

    cp_disc.wait()
    pltpu.sync_copy(disc_v, discg_hbm.at[pl.ds(base, BPW)])


BB = 2048


def _tc_body(seg_ref, dfg_ref, q_ref, disc_ref, out_ref):
    prof = jax.nn.sigmoid(seg_ref[...])
    diff = jax.nn.sigmoid(dfg_ref[...])
    s = jnp.sum((prof - diff) * q_ref[...], axis=1, keepdims=True)
    out_ref[...] = jax.nn.sigmoid(jax.nn.sigmoid(disc_ref[...]) * s)


_tc_call = pl.pallas_call(
    _tc_body,
    grid=(B // BB,),
    in_specs=[
        pl.BlockSpec((BB, D), lambda i: (i, 0)),
        pl.BlockSpec((BB, D), lambda i: (i, 0)),
        pl.BlockSpec((BB, D), lambda i: (i, 0)),
        pl.BlockSpec((BB, 1), lambda i: (i, 0)),
    ],
    out_specs=pl.BlockSpec((BB, 1), lambda i: (i, 0)),
    out_shape=jax.ShapeDtypeStruct((B, 1), jnp.float32),
)


def kernel(student_id, question, q_matrix_line, student_emb, difficulty, discrimination):
    sid = student_id.astype(jnp.int32)
    qid = question.astype(jnp.int32)
    seg, dfg, discg = _gather_sc(sid, qid, student_emb, difficulty,
                                 discrimination.reshape(-1))
    out = _tc_call(seg, dfg, q_matrix_line, discg.reshape(B, 1))
    return out.reshape(-1)
